# baseline (speedup 1.0000x reference)
import functools

import jax
import jax.numpy as jnp
from jax import lax
from jax.experimental import pallas as pl
from jax.experimental.pallas import tpu as pltpu

_K = 10
_ALPHA = 0.1
_TILE = 2000


def _body(x_ref, u_ref, v_ref, w_ref, out_ref, s_ref):
    phase = pl.program_id(0)
    i = pl.program_id(1)

    @pl.when(phase == 0)
    def _accumulate():
        @pl.when(i == 0)
        def _init():
            s_ref[...] = jnp.zeros_like(s_ref)

        s_ref[...] += lax.dot_general(
            u_ref[...].astype(jnp.bfloat16), x_ref[...].astype(jnp.bfloat16),
            (((0,), (0,)), ((), ())),
            preferred_element_type=jnp.float32,
        )

    @pl.when(phase == 1)
    def _emit():
        v = v_ref[...]
        v_pow = jnp.ones_like(v)
        v_out = jnp.zeros_like(v)
        for _ in range(_K):
            v_pow = v_pow * v
            v_out = v_out + (1.0 - _ALPHA) * v_pow
        v_out = v_out / _K

        uw = (u_ref[...] * v_out).astype(jnp.bfloat16)
        t = lax.dot_general(
            uw, s_ref[...].astype(jnp.bfloat16),
            (((1,), (0,)), ((), ())),
            preferred_element_type=jnp.float32,
        ) + _ALPHA * x_ref[...]
        out_ref[...] = lax.dot_general(
            t.astype(jnp.bfloat16), w_ref[...],
            (((1,), (1,)), ((), ())),
            preferred_element_type=jnp.float32,
        )


@functools.partial(jax.jit, static_argnames=())
def kernel(x, U, V, W):
    n, d = x.shape
    keig = U.shape[1]
    num_tiles = n // _TILE
    assert num_tiles * _TILE == n

    v2 = V.reshape(1, keig)
    w_bf = W.astype(jnp.bfloat16)

    grid = (2, num_tiles)
    out = pl.pallas_call(
        _body,
        grid=grid,
        in_specs=[
            pl.BlockSpec((_TILE, d), lambda p, i: (i, 0)),
            pl.BlockSpec((_TILE, keig), lambda p, i: (i, 0)),
            pl.BlockSpec((1, keig), lambda p, i: (0, 0)),
            pl.BlockSpec((d, d), lambda p, i: (0, 0)),
        ],
        out_specs=pl.BlockSpec((_TILE, d), lambda p, i: (p * i, 0)),
        out_shape=jax.ShapeDtypeStruct((n, d), jnp.float32),
        scratch_shapes=[pltpu.VMEM((keig, d), jnp.float32)],
        compiler_params=pltpu.CompilerParams(
            dimension_semantics=("arbitrary", "arbitrary"),
        ),
    )(x, U, v2, w_bf)
    return out


# speedup vs baseline: 1.3317x; 1.3317x over previous
import functools

import jax
import jax.numpy as jnp
from jax import lax
from jax.experimental import pallas as pl
from jax.experimental.pallas import tpu as pltpu

_K = 10
_ALPHA = 0.1
_TILE = 10000


def _sbody(x_ref, u_ref, s_ref):
    i = pl.program_id(0)

    @pl.when(i == 0)
    def _init():
        s_ref[...] = jnp.zeros_like(s_ref)

    s_ref[...] += lax.dot_general(
        u_ref[...].astype(jnp.bfloat16), x_ref[...].astype(jnp.bfloat16),
        (((0,), (0,)), ((), ())),
        preferred_element_type=jnp.float32,
    )


def _obody(x_ref, u_ref, v_ref, w_ref, s_ref, out_ref, g_ref, wa_ref):
    i = pl.program_id(0)

    @pl.when(i == 0)
    def _make_small_mats():
        v = v_ref[...]
        v_pow = jnp.ones_like(v)
        v_out = jnp.zeros_like(v)
        for _ in range(_K):
            v_pow = v_pow * v
            v_out = v_out + (1.0 - _ALPHA) * v_pow
        v_out = v_out / _K

        g32 = lax.dot_general(
            v_out * s_ref[...], w_ref[...],
            (((1,), (1,)), ((), ())),
            preferred_element_type=jnp.float32,
        )
        g_ref[...] = g32.astype(jnp.bfloat16)
        wa_ref[...] = (_ALPHA * w_ref[...]).astype(jnp.bfloat16)

    out_ref[...] = lax.dot_general(
        u_ref[...].astype(jnp.bfloat16), g_ref[...],
        (((1,), (0,)), ((), ())),
        preferred_element_type=jnp.float32,
    ) + lax.dot_general(
        x_ref[...].astype(jnp.bfloat16), wa_ref[...],
        (((1,), (1,)), ((), ())),
        preferred_element_type=jnp.float32,
    )


@functools.partial(jax.jit, static_argnames=())
def kernel(x, U, V, W):
    n, d = x.shape
    keig = U.shape[1]
    num_tiles = n // _TILE
    assert num_tiles * _TILE == n

    v2 = V.reshape(keig, 1)

    s = pl.pallas_call(
        _sbody,
        grid=(num_tiles,),
        in_specs=[
            pl.BlockSpec((_TILE, d), lambda i: (i, 0)),
            pl.BlockSpec((_TILE, keig), lambda i: (i, 0)),
        ],
        out_specs=pl.BlockSpec((keig, d), lambda i: (0, 0)),
        out_shape=jax.ShapeDtypeStruct((keig, d), jnp.float32),
        compiler_params=pltpu.CompilerParams(
            dimension_semantics=("arbitrary",),
        ),
    )(x, U)

    out = pl.pallas_call(
        _obody,
        grid=(num_tiles,),
        in_specs=[
            pl.BlockSpec((_TILE, d), lambda i: (i, 0)),
            pl.BlockSpec((_TILE, keig), lambda i: (i, 0)),
            pl.BlockSpec((keig, 1), lambda i: (0, 0)),
            pl.BlockSpec((d, d), lambda i: (0, 0)),
            pl.BlockSpec((keig, d), lambda i: (0, 0)),
        ],
        out_specs=pl.BlockSpec((_TILE, d), lambda i: (i, 0)),
        out_shape=jax.ShapeDtypeStruct((n, d), jnp.float32),
        scratch_shapes=[
            pltpu.VMEM((keig, d), jnp.bfloat16),
            pltpu.VMEM((d, d), jnp.bfloat16),
        ],
        compiler_params=pltpu.CompilerParams(
            dimension_semantics=("arbitrary",),
        ),
    )(x, U, v2, W, s)
    return out


# speedup vs baseline: 2.4084x; 1.8086x over previous
import functools

import jax
import jax.numpy as jnp
from jax import lax
from jax.experimental import pallas as pl
from jax.experimental.pallas import tpu as pltpu

_K = 10
_ALPHA = 0.1
_TILE = 12800


def _make_sbody(n):
    num_tiles = pl.cdiv(n, _TILE)
    edge = n - (num_tiles - 1) * _TILE

    def _sbody(x_ref, ut_ref, s_ref):
        i = pl.program_id(0)

        @pl.when(i == 0)
        def _init():
            s_ref[...] = jnp.zeros_like(s_ref)

        @pl.when(i < num_tiles - 1)
        def _full():
            s_ref[...] += lax.dot_general(
                ut_ref[...].astype(jnp.bfloat16),
                x_ref[...].astype(jnp.bfloat16),
                (((1,), (0,)), ((), ())),
                preferred_element_type=jnp.float32,
            )

        @pl.when(i == num_tiles - 1)
        def _edge():
            s_ref[...] += lax.dot_general(
                ut_ref[:, :edge].astype(jnp.bfloat16),
                x_ref[:edge, :].astype(jnp.bfloat16),
                (((1,), (0,)), ((), ())),
                preferred_element_type=jnp.float32,
            )

    return _sbody


def _obody(x_ref, ut_ref, v_ref, w_ref, s_ref, out_ref, g_ref, wa_ref):
    i = pl.program_id(0)

    @pl.when(i == 0)
    def _make_small_mats():
        v = v_ref[...]
        v_pow = jnp.ones_like(v)
        v_out = jnp.zeros_like(v)
        for _ in range(_K):
            v_pow = v_pow * v
            v_out = v_out + (1.0 - _ALPHA) * v_pow
        v_out = v_out / _K

        g32 = lax.dot_general(
            v_out * s_ref[...], w_ref[...],
            (((1,), (1,)), ((), ())),
            preferred_element_type=jnp.float32,
        )
        g_ref[...] = g32.astype(jnp.bfloat16)
        wa_ref[...] = (_ALPHA * w_ref[...]).astype(jnp.bfloat16)

    out_ref[...] = lax.dot_general(
        ut_ref[...].astype(jnp.bfloat16), g_ref[...],
        (((0,), (0,)), ((), ())),
        preferred_element_type=jnp.float32,
    ) + lax.dot_general(
        x_ref[...].astype(jnp.bfloat16), wa_ref[...],
        (((1,), (1,)), ((), ())),
        preferred_element_type=jnp.float32,
    )


@functools.partial(jax.jit, static_argnames=())
def kernel(x, U, V, W):
    n, d = x.shape
    keig = U.shape[1]
    num_tiles = pl.cdiv(n, _TILE)

    v2 = V.reshape(keig, 1)
    ut = U.T

    s = pl.pallas_call(
        _make_sbody(n),
        grid=(num_tiles,),
        in_specs=[
            pl.BlockSpec((_TILE, d), lambda i: (i, 0)),
            pl.BlockSpec((keig, _TILE), lambda i: (0, i)),
        ],
        out_specs=pl.BlockSpec((keig, d), lambda i: (0, 0)),
        out_shape=jax.ShapeDtypeStruct((keig, d), jnp.float32),
        compiler_params=pltpu.CompilerParams(
            dimension_semantics=("arbitrary",),
        ),
    )(x, ut)

    out = pl.pallas_call(
        _obody,
        grid=(num_tiles,),
        in_specs=[
            pl.BlockSpec((_TILE, d), lambda i: (i, 0)),
            pl.BlockSpec((keig, _TILE), lambda i: (0, i)),
            pl.BlockSpec((keig, 1), lambda i: (0, 0)),
            pl.BlockSpec((d, d), lambda i: (0, 0)),
            pl.BlockSpec((keig, d), lambda i: (0, 0)),
        ],
        out_specs=pl.BlockSpec((_TILE, d), lambda i: (i, 0)),
        out_shape=jax.ShapeDtypeStruct((n, d), jnp.float32),
        scratch_shapes=[
            pltpu.VMEM((keig, d), jnp.bfloat16),
            pltpu.VMEM((d, d), jnp.bfloat16),
        ],
        compiler_params=pltpu.CompilerParams(
            dimension_semantics=("arbitrary",),
        ),
    )(x, ut, v2, W, s)
    return out


# speedup vs baseline: 2.4807x; 1.0300x over previous
import functools

import jax
import jax.numpy as jnp
from jax import lax
from jax.experimental import pallas as pl
from jax.experimental.pallas import tpu as pltpu

_K = 10
_ALPHA = 0.1
_TILE = 12800


def _make_body(n):
    num_tiles = pl.cdiv(n, _TILE)
    edge = n - (num_tiles - 1) * _TILE

    def _body(x_ref, ut_ref, v_ref, w_ref, out_ref, s_ref, g_ref, wa_ref):
        phase = pl.program_id(0)
        i = pl.program_id(1)

        @pl.when((phase == 0) & (i == 0))
        def _init():
            s_ref[...] = jnp.zeros_like(s_ref)

        @pl.when((phase == 0) & (i < num_tiles - 1))
        def _accum_full():
            s_ref[...] += lax.dot_general(
                ut_ref[...].astype(jnp.bfloat16),
                x_ref[...].astype(jnp.bfloat16),
                (((1,), (0,)), ((), ())),
                preferred_element_type=jnp.float32,
            )

        @pl.when((phase == 0) & (i == num_tiles - 1))
        def _accum_edge_and_small_mats():
            s = s_ref[...] + lax.dot_general(
                ut_ref[:, :edge].astype(jnp.bfloat16),
                x_ref[:edge, :].astype(jnp.bfloat16),
                (((1,), (0,)), ((), ())),
                preferred_element_type=jnp.float32,
            )

            v = v_ref[...]
            v_pow = jnp.ones_like(v)
            v_out = jnp.zeros_like(v)
            for _ in range(_K):
                v_pow = v_pow * v
                v_out = v_out + (1.0 - _ALPHA) * v_pow
            v_out = v_out / _K

            g32 = lax.dot_general(
                v_out * s, w_ref[...],
                (((1,), (1,)), ((), ())),
                preferred_element_type=jnp.float32,
            )
            g_ref[...] = g32.astype(jnp.bfloat16)
            wa_ref[...] = (_ALPHA * w_ref[...]).astype(jnp.bfloat16)

        @pl.when(phase == 1)
        def _emit():
            out_ref[...] = lax.dot_general(
                ut_ref[...].astype(jnp.bfloat16), g_ref[...],
                (((0,), (0,)), ((), ())),
                preferred_element_type=jnp.float32,
            ) + lax.dot_general(
                x_ref[...].astype(jnp.bfloat16), wa_ref[...],
                (((1,), (1,)), ((), ())),
                preferred_element_type=jnp.float32,
            )

    return _body


@functools.partial(jax.jit, static_argnames=())
def kernel(x, U, V, W):
    n, d = x.shape
    keig = U.shape[1]
    num_tiles = pl.cdiv(n, _TILE)

    v2 = V.reshape(keig, 1)
    ut = U.T

    out = pl.pallas_call(
        _make_body(n),
        grid=(2, num_tiles),
        in_specs=[
            pl.BlockSpec((_TILE, d), lambda p, i: (i, 0)),
            pl.BlockSpec((keig, _TILE), lambda p, i: (0, i)),
            pl.BlockSpec((keig, 1), lambda p, i: (0, 0)),
            pl.BlockSpec((d, d), lambda p, i: (0, 0)),
        ],
        out_specs=pl.BlockSpec((_TILE, d), lambda p, i: (p * i, 0)),
        out_shape=jax.ShapeDtypeStruct((n, d), jnp.float32),
        scratch_shapes=[
            pltpu.VMEM((keig, d), jnp.float32),
            pltpu.VMEM((keig, d), jnp.bfloat16),
            pltpu.VMEM((d, d), jnp.bfloat16),
        ],
        compiler_params=pltpu.CompilerParams(
            dimension_semantics=("arbitrary", "arbitrary"),
        ),
    )(x, ut, v2, W)
    return out
